# baseline (speedup 1.0000x reference)
import functools

import jax
import jax.numpy as jnp
from jax import lax
from jax.experimental import pallas as pl
from jax.experimental.pallas import tpu as pltpu
from jax.experimental.pallas import tpu_sc as plsc

_NUM_CORES = 2
_NUM_SUBCORES = 16
_NUM_WORKERS = _NUM_CORES * _NUM_SUBCORES
_CHUNK = 128
_NBUF = 8
_INFLIGHT = 4


@functools.partial(jax.jit, static_argnums=(2, 3))
def _sc_gather(table, idx, n_chunks_w, d):
    nbuf = _NBUF
    lead = _INFLIGHT
    n_rounds = n_chunks_w // nbuf

    scratch = [
        pltpu.VMEM((n_chunks_w, _CHUNK), jnp.int32),
        pltpu.VMEM((nbuf, _CHUNK, d), jnp.float32),
    ]
    scratch += [pltpu.SemaphoreType.DMA] * (2 * nbuf)

    @functools.partial(
        pl.kernel,
        mesh=plsc.VectorSubcoreMesh(core_axis_name="c", subcore_axis_name="s"),
        out_type=jax.ShapeDtypeStruct(
            (_NUM_WORKERS * n_chunks_w, _CHUNK, d), jnp.float32
        ),
        scratch_types=scratch,
        compiler_params=pltpu.CompilerParams(use_tc_tiling_on_sc=False),
    )
    def body(table_hbm, idx_hbm, out_hbm, idx_v, rows_v, *sems):
        gsems = sems[:nbuf]
        ssems = sems[nbuf:]
        wid = lax.axis_index("s") * _NUM_CORES + lax.axis_index("c")
        base = wid * n_chunks_w

        pltpu.sync_copy(idx_hbm.at[wid], idx_v)

        def gather(c, slot):
            return pltpu.make_async_copy(
                table_hbm.at[idx_v.at[c]], rows_v.at[slot], gsems[slot]
            )

        def scatter(c, slot):
            return pltpu.make_async_copy(
                rows_v.at[slot], out_hbm.at[base + c], ssems[slot]
            )

        for slot in range(lead):
            gather(slot, slot).start()

        for b in range(nbuf):
            gather(b, b).wait()
            scatter(b, b).start()
            if b >= lead:
                scatter(b - lead, b - lead).wait()
            gather(b + lead, (b + lead) % nbuf).start()

        def round_body(r, carry):
            c0 = r * nbuf
            for b in range(nbuf):
                c = c0 + b
                gather(c, b).wait()
                scatter(c, b).start()
                scatter(c - lead, (b + lead) % nbuf).wait()
                gather(c + lead, (b + lead) % nbuf).start()
            return carry

        lax.fori_loop(1, n_rounds - 1, round_body, 0)

        c0 = (n_rounds - 1) * nbuf
        for b in range(nbuf):
            c = c0 + b
            gather(c, b).wait()
            scatter(c, b).start()
            if b < lead:
                scatter(c - lead, (b + lead) % nbuf).wait()
                gather(c + lead, (b + lead) % nbuf).start()
        for b in range(nbuf):
            scatter(c0 + b, b).wait()

    return body(table, idx)


def kernel(tokens, embedding_weight):
    bsz, seq = tokens.shape
    _, d = embedding_weight.shape
    n = bsz * seq
    span = _NUM_WORKERS * _CHUNK * _NBUF
    n_pad = -(-n // span) * span
    idx = tokens.astype(jnp.int32).reshape(-1)
    if n_pad != n:
        idx = jnp.pad(idx, (0, n_pad - n))
    n_chunks_w = n_pad // (_NUM_WORKERS * _CHUNK)
    idx = idx.reshape(_NUM_WORKERS, n_chunks_w, _CHUNK)
    out = _sc_gather(embedding_weight, idx, n_chunks_w, d)
    out = out.reshape(n_pad, d)[:n]
    return out.reshape(bsz, seq, d)
